# baseline (speedup 1.0000x reference)
import functools

import jax
import jax.numpy as jnp
from jax import lax
from jax.experimental import pallas as pl
from jax.experimental.pallas import tpu as pltpu
from jax.experimental.pallas import tpu_sc as plsc

N = 10000
E = 320000
F = 128
H = 256

BN = 256
NP = 10240
NBLK = NP // BN

DT = 128
NW = 32
CH = 64
NCH = 160
EPAD = NW * NCH * CH

_f32 = jnp.float32
_INTERPRET = False


def _leaky(x):
    return jnp.where(x > 0, x, 0.1 * x)


def _prep_body(delay_ref, feat_ref, pw1_ref, pb1_ref, cfold_ref, c0_ref,
               sw1t_ref, sb1_ref, sw2t_ref, sb2_ref,
               table_ref, dmax_ref, hs_ref):
    d = delay_ref[...]
    z = d * pw1_ref[...] + pb1_ref[...]
    mid = _leaky(z)
    table_ref[...] = (
        jnp.dot(mid, cfold_ref[...], preferred_element_type=_f32) + c0_ref[...]
    )
    i = pl.program_id(0)
    m = jnp.max(d, keepdims=True)
    prev = jnp.where(i == 0, jnp.full((1, 1), -jnp.inf, _f32), dmax_ref[...])
    dmax_ref[...] = jnp.maximum(prev, m)
    s1 = _leaky(
        jnp.dot(feat_ref[...], sw1t_ref[...], preferred_element_type=_f32)
        + sb1_ref[...]
    )
    hs_ref[...] = (
        jnp.dot(s1, sw2t_ref[...], preferred_element_type=_f32) + sb2_ref[...]
    )


def _prep_call(delay_p, feat_p, pw1, pb1, cfold, c0, sw1t, sb1, sw2t, sb2):
    full = lambda r, c: pl.BlockSpec((r, c), lambda i: (0, 0))
    return pl.pallas_call(
        _prep_body,
        grid=(NBLK,),
        in_specs=[
            pl.BlockSpec((BN, 1), lambda i: (i, 0)),
            pl.BlockSpec((BN, F), lambda i: (i, 0)),
            full(1, 128), full(1, 128),
            full(128, 128), full(1, 128),
            full(F, 128), full(1, 128), full(128, H), full(1, H),
        ],
        out_specs=[
            pl.BlockSpec((BN, DT), lambda i: (i, 0)),
            pl.BlockSpec((1, 1), lambda i: (0, 0)),
            pl.BlockSpec((BN, H), lambda i: (i, 0)),
        ],
        out_shape=[
            jax.ShapeDtypeStruct((NP, DT), _f32),
            jax.ShapeDtypeStruct((1, 1), _f32),
            jax.ShapeDtypeStruct((NP, H), _f32),
        ],
        interpret=_INTERPRET,
    )(delay_p, feat_p, pw1, pb1, cfold, c0, sw1t, sb1, sw2t, sb2)


NB = 4
NPH = 4
HCH = NCH // NPH
NGRP = HCH // NB


def _seg_body(table_hbm, src_hbm, dst_hbm, zeros_hbm, out_hbm,
              src_v, dst_v, rows_v, acc_sh,
              gs0, gs1, gs2, gs3, ss0, ss1, ss2, ss3):
    gsems = (gs0, gs1, gs2, gs3)
    ssems = (ss0, ss1, ss2, ss3)
    cid = lax.axis_index("c")
    sid = lax.axis_index("s")
    wid = cid * 16 + sid
    rpt = NP // 16
    pltpu.sync_copy(zeros_hbm.at[pl.ds(sid * rpt, rpt)],
                    acc_sh.at[pl.ds(sid * rpt, rpt)])
    plsc.subcore_barrier()

    def gst(j, b):
        pltpu.async_copy(table_hbm.at[src_v.at[j]], rows_v.at[b], gsems[b])

    def gwt(j, b):
        pltpu.make_async_copy(table_hbm.at[src_v.at[j]], rows_v.at[b],
                              gsems[b]).wait()

    def sst(j, b):
        pltpu.async_copy(rows_v.at[b], acc_sh.at[dst_v.at[j]], ssems[b],
                         add=True)

    def swt(j, b):
        pltpu.make_async_copy(rows_v.at[b], acc_sh.at[dst_v.at[j]],
                              ssems[b]).wait()

    for phase in range(NPH):
        base = wid * NCH + phase * HCH
        pltpu.sync_copy(src_hbm.at[pl.ds(base, HCH)], src_v)
        pltpu.sync_copy(dst_hbm.at[pl.ds(base, HCH)], dst_v)
        for b in range(NB):
            gst(b, b)

        def body(i, carry):
            for b in range(NB):
                j = i * NB + b
                gwt(j, b)
                sst(j, b)
            for b in range(NB):
                j = i * NB + b
                swt(j, b)

                @pl.when(i < NGRP - 1)
                def _(i=i, b=b, j=j):
                    gst(j + NB, b)
            return carry

        lax.fori_loop(0, NGRP, body, 0)

    plsc.subcore_barrier()
    pltpu.sync_copy(acc_sh.at[pl.ds(sid * rpt, rpt)],
                    out_hbm.at[cid, pl.ds(sid * rpt, rpt)])


def _seg_call(table, src_p, dst_p, zeros_hbm):
    return pl.kernel(
        _seg_body,
        out_type=jax.ShapeDtypeStruct((2, NP, DT), _f32),
        mesh=plsc.VectorSubcoreMesh(core_axis_name="c", subcore_axis_name="s",
                                    num_cores=2, num_subcores=16),
        scratch_types=[
            pltpu.VMEM((HCH, CH), jnp.int32),
            pltpu.VMEM((HCH, CH), jnp.int32),
            pltpu.VMEM((NB, CH, DT), _f32),
            pltpu.VMEM_SHARED((NP, DT), _f32),
            pltpu.SemaphoreType.DMA, pltpu.SemaphoreType.DMA,
            pltpu.SemaphoreType.DMA, pltpu.SemaphoreType.DMA,
            pltpu.SemaphoreType.DMA, pltpu.SemaphoreType.DMA,
            pltpu.SemaphoreType.DMA, pltpu.SemaphoreType.DMA,
        ],
    )(table, src_p, dst_p, zeros_hbm)


def _dense_body(accs_ref, hs_ref, ispo_ref, dmax_ref,
                nb1_ref, nw2t_ref, nb2_ref,
                gw1_ref, gb1_ref, gw2t_ref, gb2_ref,
                ow1at_ref, ow1bt_ref, ob1_ref, ow2t_ref, ob2_ref,
                out_ref):
    acc = accs_ref[0] + accs_ref[1]
    n1 = _leaky(acc + nb1_ref[...])
    hn = jnp.dot(n1, nw2t_ref[...], preferred_element_type=_f32) + nb2_ref[...]
    h = hn + hs_ref[...]
    mask = ispo_ref[...] != 1
    h = jnp.where(mask, jnp.maximum(h, 0.0), h)
    dmax = dmax_ref[...]
    g1 = _leaky(dmax * gw1_ref[...] + gb1_ref[...])
    g = jnp.dot(g1, gw2t_ref[...], preferred_element_type=_f32) + gb2_ref[...]
    r = jnp.dot(g, ow1bt_ref[...], preferred_element_type=_f32)
    o1 = _leaky(
        jnp.dot(h, ow1at_ref[...], preferred_element_type=_f32)
        + r + ob1_ref[...]
    )
    out_ref[...] = (
        jnp.dot(o1, ow2t_ref[...], preferred_element_type=_f32) + ob2_ref[...]
    )


def _dense_call(accs, hs, ispo_p, dmax, nb1, nw2t, nb2,
                gw1, gb1, gw2t, gb2, ow1at, ow1bt, ob1, ow2t, ob2):
    full = lambda r, c: pl.BlockSpec((r, c), lambda i: (0, 0))
    return pl.pallas_call(
        _dense_body,
        grid=(NBLK,),
        in_specs=[
            pl.BlockSpec((2, BN, DT), lambda i: (0, i, 0)),
            pl.BlockSpec((BN, H), lambda i: (i, 0)),
            pl.BlockSpec((BN, 1), lambda i: (i, 0)),
            full(1, 1),
            full(1, 128), full(128, H), full(1, H),
            full(1, 128), full(1, 128), full(128, H), full(1, H),
            full(H, H), full(H, H), full(1, H), full(H, 1), full(1, 1),
        ],
        out_specs=pl.BlockSpec((BN, 1), lambda i: (i, 0)),
        out_shape=jax.ShapeDtypeStruct((NP, 1), _f32),
        interpret=_INTERPRET,
    )(accs, hs, ispo_p, dmax, nb1, nw2t, nb2,
      gw1, gb1, gw2t, gb2, ow1at, ow1bt, ob1, ow2t, ob2)


def kernel(feat, delay, edge_index, is_po,
           pi_W1, pi_b1, pi_W2, pi_b2,
           self_W1, self_b1, self_W2, self_b2,
           neigh_W1, neigh_b1, neigh_W2, neigh_b2,
           glob_W1, glob_b1, glob_W2, glob_b2,
           out_W1, out_b1, out_W2, out_b2):
    delay_p = jnp.pad(delay.astype(_f32), ((0, NP - N), (0, 0)),
                      constant_values=-1e30)
    feat_p = jnp.pad(feat.astype(_f32), ((0, NP - N), (0, 0)))
    ispo_p = jnp.pad(is_po.astype(jnp.int32).reshape(N, 1),
                     ((0, NP - N), (0, 0)))
    src_p = jnp.pad(edge_index[0].astype(jnp.int32),
                    (0, EPAD - E)).reshape(NW * NCH, CH)
    dst_p = jnp.pad(edge_index[1].astype(jnp.int32), (0, EPAD - E),
                    constant_values=N).reshape(NW * NCH, CH)
    zeros_hbm = jnp.zeros((NP, DT), _f32)

    pw1 = pi_W1.reshape(1, 128)
    pb1 = pi_b1.reshape(1, 128)
    cfold = pi_W2.T @ neigh_W1.T
    c0 = (pi_b2 @ neigh_W1.T).reshape(1, 128)
    sw1t = self_W1.T
    sb1 = self_b1.reshape(1, 128)
    sw2t = self_W2.T
    sb2 = self_b2.reshape(1, H)
    nb1 = neigh_b1.reshape(1, 128)
    nw2t = neigh_W2.T
    nb2 = neigh_b2.reshape(1, H)
    gw1 = glob_W1.reshape(1, 128)
    gb1 = glob_b1.reshape(1, 128)
    gw2t = glob_W2.T
    gb2 = glob_b2.reshape(1, H)
    ow1at = out_W1[:, :H].T
    ow1bt = out_W1[:, H:].T
    ob1 = out_b1.reshape(1, H)
    ow2t = out_W2.T
    ob2 = out_b2.reshape(1, 1)

    table, dmax, hs = _prep_call(delay_p, feat_p, pw1, pb1, cfold, c0,
                                 sw1t, sb1, sw2t, sb2)
    accs = _seg_call(table, src_p, dst_p, zeros_hbm)
    out_p = _dense_call(accs, hs, ispo_p, dmax, nb1,
                        nw2t, nb2, gw1, gb1, gw2t, gb2,
                        ow1at, ow1bt, ob1, ow2t, ob2)
    return out_p[:N]


# speedup vs baseline: 2.3076x; 2.3076x over previous
import jax
import jax.numpy as jnp
from jax import lax
from jax.experimental import pallas as pl
from jax.experimental.pallas import tpu as pltpu
from jax.experimental.pallas import tpu_sc as plsc

N = 10000
E = 320000
F = 128
H = 256

BN = 256
NP = 10240
NBLK = NP // BN

DT = 128
NW = 32
CH = 125
NCH = 80
EPAD = NW * NCH * CH

_f32 = jnp.float32


def _leaky(x):
    return jnp.where(x > 0, x, 0.1 * x)


def _prep_body(delay_ref, feat_ref, pw1_ref, pb1_ref, cfold_ref, c0_ref,
               sw1t_ref, sb1_ref, sw2t_ref, sb2_ref,
               table_ref, dmax_ref, hs_ref):
    d = delay_ref[...]
    z = d * pw1_ref[...] + pb1_ref[...]
    mid = _leaky(z)
    table_ref[...] = (
        jnp.dot(mid, cfold_ref[...], preferred_element_type=_f32, precision=lax.Precision.HIGHEST) + c0_ref[...]
    )
    i = pl.program_id(0)
    m = jnp.max(d, keepdims=True)
    prev = jnp.where(i == 0, jnp.full((1, 1), -jnp.inf, _f32), dmax_ref[...])
    dmax_ref[...] = jnp.maximum(prev, m)
    s1 = _leaky(
        jnp.dot(feat_ref[...], sw1t_ref[...], preferred_element_type=_f32, precision=lax.Precision.HIGHEST)
        + sb1_ref[...]
    )
    hs_ref[...] = (
        jnp.dot(s1, sw2t_ref[...], preferred_element_type=_f32, precision=lax.Precision.HIGHEST) + sb2_ref[...]
    )


def _prep_call(delay_p, feat_p, pw1, pb1, cfold, c0, sw1t, sb1, sw2t, sb2):
    full = lambda r, c: pl.BlockSpec((r, c), lambda i: (0, 0))
    return pl.pallas_call(
        _prep_body,
        grid=(NBLK,),
        in_specs=[
            pl.BlockSpec((BN, 1), lambda i: (i, 0)),
            pl.BlockSpec((BN, F), lambda i: (i, 0)),
            full(1, 128), full(1, 128),
            full(128, 128), full(1, 128),
            full(F, 128), full(1, 128), full(128, H), full(1, H),
        ],
        out_specs=[
            pl.BlockSpec((BN, DT), lambda i: (i, 0)),
            pl.BlockSpec((1, 1), lambda i: (0, 0)),
            pl.BlockSpec((BN, H), lambda i: (i, 0)),
        ],
        out_shape=[
            jax.ShapeDtypeStruct((NP, DT), _f32),
            jax.ShapeDtypeStruct((1, 1), _f32),
            jax.ShapeDtypeStruct((NP, H), _f32),
        ],
    )(delay_p, feat_p, pw1, pb1, cfold, c0, sw1t, sb1, sw2t, sb2)


NB = 2
NPH = 2
HCH = NCH // NPH


def _seg_body(table_hbm, src_hbm, dst_hbm, zeros_hbm, out_hbm,
              src_v, dst_v, rows_v, acc_sh, gs0, gs1):
    gsems = (gs0, gs1)
    cid = lax.axis_index("c")
    sid = lax.axis_index("s")
    wid = cid * 16 + sid
    rpt = NP // 16
    for r in range(rpt // 128):
        pltpu.sync_copy(zeros_hbm,
                        acc_sh.at[pl.ds(sid * rpt + r * 128, 128)])
    plsc.subcore_barrier()

    def gst(j, b):
        pltpu.async_copy(table_hbm.at[src_v.at[j]], rows_v.at[b], gsems[b])

    def gwt(j, b):
        pltpu.make_async_copy(table_hbm.at[src_v.at[j]], rows_v.at[b],
                              gsems[b]).wait()

    for phase in range(NPH):
        base = wid * NCH + phase * HCH
        pltpu.sync_copy(src_hbm.at[pl.ds(base, HCH)], src_v)
        pltpu.sync_copy(dst_hbm.at[pl.ds(base, HCH)], dst_v)
        gst(0, 0)

        def body(i, carry):
            j0 = 2 * i
            j1 = 2 * i + 1
            gst(j1, 1)
            gwt(j0, 0)
            pltpu.sync_copy(rows_v.at[0], acc_sh.at[dst_v.at[j0]], add=True)

            @pl.when(j1 + 1 < HCH)
            def _():
                gst(j1 + 1, 0)

            gwt(j1, 1)
            pltpu.sync_copy(rows_v.at[1], acc_sh.at[dst_v.at[j1]], add=True)
            return carry

        lax.fori_loop(0, HCH // 2, body, 0)

    plsc.subcore_barrier()
    pltpu.sync_copy(acc_sh.at[pl.ds(sid * rpt, rpt)],
                    out_hbm.at[cid, pl.ds(sid * rpt, rpt)])


def _seg_call(table, src_p, dst_p, zeros_hbm):
    return pl.kernel(
        _seg_body,
        out_type=jax.ShapeDtypeStruct((2, NP, DT), _f32),
        mesh=plsc.VectorSubcoreMesh(core_axis_name="c", subcore_axis_name="s",
                                    num_cores=2, num_subcores=16),
        scratch_types=[
            pltpu.VMEM((HCH, CH), jnp.int32),
            pltpu.VMEM((HCH, CH), jnp.int32),
            pltpu.VMEM((NB, CH, DT), _f32),
            pltpu.VMEM_SHARED((NP, DT), _f32),
            pltpu.SemaphoreType.DMA, pltpu.SemaphoreType.DMA,
        ],
    )(table, src_p, dst_p, zeros_hbm)


def _dense_body(accs_ref, hs_ref, ispo_ref, dmax_ref,
                nb1_ref, nw2t_ref, nb2_ref,
                gw1_ref, gb1_ref, gw2t_ref, gb2_ref,
                ow1at_ref, ow1bt_ref, ob1_ref, ow2t_ref, ob2_ref,
                out_ref):
    acc = accs_ref[0] + accs_ref[1]
    n1 = _leaky(acc + nb1_ref[...])
    hn = jnp.dot(n1, nw2t_ref[...], preferred_element_type=_f32, precision=lax.Precision.HIGHEST) + nb2_ref[...]
    h = hn + hs_ref[...]
    mask = ispo_ref[...] != 1
    h = jnp.where(mask, jnp.maximum(h, 0.0), h)
    dmax = dmax_ref[...]
    g1 = _leaky(dmax * gw1_ref[...] + gb1_ref[...])
    g = jnp.dot(g1, gw2t_ref[...], preferred_element_type=_f32, precision=lax.Precision.HIGHEST) + gb2_ref[...]
    r = jnp.dot(g, ow1bt_ref[...], preferred_element_type=_f32, precision=lax.Precision.HIGHEST)
    o1 = _leaky(
        jnp.dot(h, ow1at_ref[...], preferred_element_type=_f32, precision=lax.Precision.HIGHEST)
        + r + ob1_ref[...]
    )
    out_ref[...] = (
        jnp.dot(o1, ow2t_ref[...], preferred_element_type=_f32, precision=lax.Precision.HIGHEST) + ob2_ref[...]
    )


def _dense_call(accs, hs, ispo_p, dmax, nb1, nw2t, nb2,
                gw1, gb1, gw2t, gb2, ow1at, ow1bt, ob1, ow2t, ob2):
    full = lambda r, c: pl.BlockSpec((r, c), lambda i: (0, 0))
    return pl.pallas_call(
        _dense_body,
        grid=(NBLK,),
        in_specs=[
            pl.BlockSpec((2, BN, DT), lambda i: (0, i, 0)),
            pl.BlockSpec((BN, H), lambda i: (i, 0)),
            pl.BlockSpec((BN, 1), lambda i: (i, 0)),
            full(1, 1),
            full(1, 128), full(128, H), full(1, H),
            full(1, 128), full(1, 128), full(128, H), full(1, H),
            full(H, H), full(H, H), full(1, H), full(H, 1), full(1, 1),
        ],
        out_specs=pl.BlockSpec((BN, 1), lambda i: (i, 0)),
        out_shape=jax.ShapeDtypeStruct((NP, 1), _f32),
    )(accs, hs, ispo_p, dmax, nb1, nw2t, nb2,
      gw1, gb1, gw2t, gb2, ow1at, ow1bt, ob1, ow2t, ob2)


def kernel(feat, delay, edge_index, is_po,
           pi_W1, pi_b1, pi_W2, pi_b2,
           self_W1, self_b1, self_W2, self_b2,
           neigh_W1, neigh_b1, neigh_W2, neigh_b2,
           glob_W1, glob_b1, glob_W2, glob_b2,
           out_W1, out_b1, out_W2, out_b2):
    delay_p = jnp.pad(delay.astype(_f32), ((0, NP - N), (0, 0)),
                      constant_values=-1e30)
    feat_p = jnp.pad(feat.astype(_f32), ((0, NP - N), (0, 0)))
    ispo_p = jnp.pad(is_po.astype(jnp.int32).reshape(N, 1),
                     ((0, NP - N), (0, 0)))
    src_p = edge_index[0].astype(jnp.int32).reshape(NW * NCH, CH)
    dst_p = edge_index[1].astype(jnp.int32).reshape(NW * NCH, CH)
    zeros_hbm = jnp.zeros((128, DT), _f32)

    pw1 = pi_W1.reshape(1, 128)
    pb1 = pi_b1.reshape(1, 128)
    cfold = pi_W2.T @ neigh_W1.T
    c0 = (pi_b2 @ neigh_W1.T).reshape(1, 128)
    sw1t = self_W1.T
    sb1 = self_b1.reshape(1, 128)
    sw2t = self_W2.T
    sb2 = self_b2.reshape(1, H)
    nb1 = neigh_b1.reshape(1, 128)
    nw2t = neigh_W2.T
    nb2 = neigh_b2.reshape(1, H)
    gw1 = glob_W1.reshape(1, 128)
    gb1 = glob_b1.reshape(1, 128)
    gw2t = glob_W2.T
    gb2 = glob_b2.reshape(1, H)
    ow1at = out_W1[:, :H].T
    ow1bt = out_W1[:, H:].T
    ob1 = out_b1.reshape(1, H)
    ow2t = out_W2.T
    ob2 = out_b2.reshape(1, 1)

    table, dmax, hs = _prep_call(delay_p, feat_p, pw1, pb1, cfold, c0,
                                 sw1t, sb1, sw2t, sb2)
    accs = _seg_call(table, src_p, dst_p, zeros_hbm)
    out_p = _dense_call(accs, hs, ispo_p, dmax, nb1,
                        nw2t, nb2, gw1, gb1, gw2t, gb2,
                        ow1at, ow1bt, ob1, ow2t, ob2)
    return out_p[:N]
